# Initial kernel scaffold; baseline (speedup 1.0000x reference)
import jax
import jax.numpy as jnp
from jax.experimental import pallas as pl


def kernel(x, table):
    raise NotImplementedError("write your pallas kernel here")


# speedup vs baseline: 2.5985x; 2.5985x over previous
import functools

import jax
import jax.numpy as jnp
from jax import lax
from jax.experimental import pallas as pl
from jax.experimental.pallas import tpu as pltpu
from jax.experimental.pallas import tpu_sc as plsc

D_MODEL = 64
IDX_MINOR = 128
ROWS_PER_CHUNK = 4


@functools.lru_cache(maxsize=None)
def _make_kernel(B, V):
    info = plsc.get_sparse_core_info()
    nc, ns = info.num_cores, info.num_subcores
    nw = nc * ns
    n_idx_rows = B // IDX_MINOR
    rows_per_w = n_idx_rows // nw
    n_chunks = rows_per_w // ROWS_PER_CHUNK
    chunk = ROWS_PER_CHUNK * IDX_MINOR

    mesh = plsc.VectorSubcoreMesh(core_axis_name="c", subcore_axis_name="s")

    @functools.partial(
        pl.kernel,
        mesh=mesh,
        out_type=jax.ShapeDtypeStruct((B, D_MODEL), jnp.float32),
        scratch_types=[
            pltpu.VMEM((ROWS_PER_CHUNK, IDX_MINOR), jnp.int32),
            pltpu.VMEM((chunk, D_MODEL), jnp.float32),
            pltpu.SemaphoreType.DMA,
        ],
        compiler_params=pltpu.CompilerParams(use_tc_tiling_on_sc=False),
    )
    def k(x_hbm, table_hbm, out_hbm, idx_v, rows_v, sem):
        wid = lax.axis_index("s") * nc + lax.axis_index("c")
        row0 = wid * rows_per_w

        def body(g, carry):
            r = row0 + g * ROWS_PER_CHUNK
            pltpu.sync_copy(x_hbm.at[pl.ds(r, ROWS_PER_CHUNK)], idx_v)
            cps = [
                pltpu.async_copy(
                    table_hbm.at[idx_v.at[j]],
                    rows_v.at[pl.ds(j * IDX_MINOR, IDX_MINOR)],
                    sem,
                )
                for j in range(ROWS_PER_CHUNK)
            ]
            for cp in cps:
                cp.wait()
            pltpu.sync_copy(rows_v, out_hbm.at[pl.ds(r * IDX_MINOR, chunk)])
            return carry

        lax.fori_loop(0, n_chunks, body, 0)

    return k


def kernel(x, table):
    b, s = x.shape
    total = b * s
    x2d = x.reshape(total // IDX_MINOR, IDX_MINOR).astype(jnp.int32)
    out = _make_kernel(total, table.shape[0])(x2d, table)
    return out.reshape(b, s, D_MODEL)


# speedup vs baseline: 2.6003x; 1.0007x over previous
import functools

import jax
import jax.numpy as jnp
from jax import lax
from jax.experimental import pallas as pl
from jax.experimental.pallas import tpu as pltpu
from jax.experimental.pallas import tpu_sc as plsc

D_MODEL = 64
IDX_MINOR = 128
ROWS_PER_CHUNK = 4


@functools.lru_cache(maxsize=None)
def _make_kernel(B, V):
    info = plsc.get_sparse_core_info()
    nc, ns = info.num_cores, info.num_subcores
    nw = nc * ns
    n_idx_rows = B // IDX_MINOR
    rows_per_w = n_idx_rows // nw
    n_chunks = rows_per_w // ROWS_PER_CHUNK
    chunk = ROWS_PER_CHUNK * IDX_MINOR
    assert n_chunks % 2 == 0

    mesh = plsc.VectorSubcoreMesh(core_axis_name="c", subcore_axis_name="s")

    @functools.partial(
        pl.kernel,
        mesh=mesh,
        out_type=jax.ShapeDtypeStruct((B, D_MODEL), jnp.float32),
        scratch_types=[
            pltpu.VMEM((ROWS_PER_CHUNK, IDX_MINOR), jnp.int32),
            pltpu.VMEM((ROWS_PER_CHUNK, IDX_MINOR), jnp.int32),
            pltpu.VMEM((chunk, D_MODEL), jnp.float32),
            pltpu.VMEM((chunk, D_MODEL), jnp.float32),
            pltpu.SemaphoreType.DMA,
            pltpu.SemaphoreType.DMA,
            pltpu.SemaphoreType.DMA,
            pltpu.SemaphoreType.DMA,
        ],
        compiler_params=pltpu.CompilerParams(use_tc_tiling_on_sc=False),
    )
    def k(x_hbm, table_hbm, out_hbm, idx_v0, idx_v1, rows_v0, rows_v1,
          isem, gsem, osem0, osem1):
        wid = lax.axis_index("s") * nc + lax.axis_index("c")
        row0 = wid * rows_per_w

        idx_bufs = (idx_v0, idx_v1)
        rows_bufs = (rows_v0, rows_v1)
        osems = (osem0, osem1)

        pltpu.async_copy(x_hbm.at[pl.ds(row0, ROWS_PER_CHUNK)], idx_v0, isem)

        def body(s, carry):
            for b in range(2):
                g = 2 * s + b
                r = row0 + g * ROWS_PER_CHUNK
                idx_b, rows_b, osem_b = idx_bufs[b], rows_bufs[b], osems[b]
                pltpu.make_async_copy(
                    x_hbm.at[pl.ds(r, ROWS_PER_CHUNK)], idx_b, isem).wait()
                nxt = jnp.minimum(g + 1, n_chunks - 1)
                pltpu.async_copy(
                    x_hbm.at[pl.ds(row0 + nxt * ROWS_PER_CHUNK,
                                   ROWS_PER_CHUNK)],
                    idx_bufs[1 - b], isem)

                @pl.when(s > 0)
                def _():
                    pltpu.make_async_copy(
                        rows_b, out_hbm.at[pl.ds(0, chunk)], osem_b).wait()

                cps = [
                    pltpu.async_copy(
                        table_hbm.at[idx_b.at[j]],
                        rows_b.at[pl.ds(j * IDX_MINOR, IDX_MINOR)],
                        gsem,
                    )
                    for j in range(ROWS_PER_CHUNK)
                ]
                for cp in cps:
                    cp.wait()

                pltpu.async_copy(
                    rows_b, out_hbm.at[pl.ds(r * IDX_MINOR, chunk)], osem_b)
            return carry

        lax.fori_loop(0, n_chunks // 2, body, 0)

        pltpu.make_async_copy(
            rows_v0, out_hbm.at[pl.ds(0, chunk)], osem0).wait()
        pltpu.make_async_copy(
            rows_v1, out_hbm.at[pl.ds(0, chunk)], osem1).wait()
        pltpu.make_async_copy(
            x_hbm.at[pl.ds(row0, ROWS_PER_CHUNK)], idx_v0, isem).wait()

    return k


def kernel(x, table):
    b, s = x.shape
    total = b * s
    x2d = x.reshape(total // IDX_MINOR, IDX_MINOR).astype(jnp.int32)
    out = _make_kernel(total, table.shape[0])(x2d, table)
    return out.reshape(b, s, D_MODEL)


# speedup vs baseline: 5.0298x; 1.9343x over previous
import functools

import jax
import jax.numpy as jnp
from jax import lax
from jax.experimental import pallas as pl
from jax.experimental.pallas import tpu as pltpu
from jax.experimental.pallas import tpu_sc as plsc

D_MODEL = 64
IDX_MINOR = 128
ROWS_PER_CHUNK = 4


@functools.lru_cache(maxsize=None)
def _make_kernel(B, V):
    info = plsc.get_sparse_core_info()
    nc, ns = info.num_cores, info.num_subcores
    nw = nc * ns
    n_idx_rows = B // IDX_MINOR
    rows_per_w = n_idx_rows // nw
    n_chunks = rows_per_w // ROWS_PER_CHUNK
    chunk = ROWS_PER_CHUNK * IDX_MINOR
    assert n_chunks % 2 == 0

    mesh = plsc.VectorSubcoreMesh(core_axis_name="c", subcore_axis_name="s")

    @functools.partial(
        pl.kernel,
        mesh=mesh,
        out_type=jax.ShapeDtypeStruct((B, D_MODEL), jnp.float32),
        scratch_types=[
            pltpu.VMEM((ROWS_PER_CHUNK, IDX_MINOR), jnp.int32),
            pltpu.VMEM((ROWS_PER_CHUNK, IDX_MINOR), jnp.int32),
            pltpu.VMEM((chunk, D_MODEL), jnp.float32),
            pltpu.VMEM((chunk, D_MODEL), jnp.float32),
            pltpu.VMEM_SHARED((V, D_MODEL), jnp.float32),
            pltpu.SemaphoreType.DMA,
            pltpu.SemaphoreType.DMA,
            pltpu.SemaphoreType.DMA,
            pltpu.SemaphoreType.DMA,
        ],
        compiler_params=pltpu.CompilerParams(use_tc_tiling_on_sc=False),
    )
    def k(x_hbm, table_hbm, out_hbm, idx_v0, idx_v1, rows_v0, rows_v1,
          table_v, isem, gsem, osem0, osem1):
        wid = lax.axis_index("s") * nc + lax.axis_index("c")
        row0 = wid * rows_per_w

        @pl.when(lax.axis_index("s") == 0)
        def _():
            pltpu.sync_copy(table_hbm, table_v)

        plsc.subcore_barrier()

        idx_bufs = (idx_v0, idx_v1)
        rows_bufs = (rows_v0, rows_v1)
        osems = (osem0, osem1)

        pltpu.async_copy(x_hbm.at[pl.ds(row0, ROWS_PER_CHUNK)], idx_v0, isem)

        def body(s, carry):
            for b in range(2):
                g = 2 * s + b
                r = row0 + g * ROWS_PER_CHUNK
                idx_b, rows_b, osem_b = idx_bufs[b], rows_bufs[b], osems[b]
                pltpu.make_async_copy(
                    x_hbm.at[pl.ds(r, ROWS_PER_CHUNK)], idx_b, isem).wait()
                nxt = jnp.minimum(g + 1, n_chunks - 1)
                pltpu.async_copy(
                    x_hbm.at[pl.ds(row0 + nxt * ROWS_PER_CHUNK,
                                   ROWS_PER_CHUNK)],
                    idx_bufs[1 - b], isem)

                @pl.when(s > 0)
                def _():
                    pltpu.make_async_copy(
                        rows_b, out_hbm.at[pl.ds(0, chunk)], osem_b).wait()

                cps = [
                    pltpu.async_copy(
                        table_v.at[idx_b.at[j]],
                        rows_b.at[pl.ds(j * IDX_MINOR, IDX_MINOR)],
                        gsem,
                    )
                    for j in range(ROWS_PER_CHUNK)
                ]
                for cp in cps:
                    cp.wait()

                pltpu.async_copy(
                    rows_b, out_hbm.at[pl.ds(r * IDX_MINOR, chunk)], osem_b)
            return carry

        lax.fori_loop(0, n_chunks // 2, body, 0)

        pltpu.make_async_copy(
            rows_v0, out_hbm.at[pl.ds(0, chunk)], osem0).wait()
        pltpu.make_async_copy(
            rows_v1, out_hbm.at[pl.ds(0, chunk)], osem1).wait()
        pltpu.make_async_copy(
            x_hbm.at[pl.ds(row0, ROWS_PER_CHUNK)], idx_v0, isem).wait()

    return k


def kernel(x, table):
    b, s = x.shape
    total = b * s
    x2d = x.reshape(total // IDX_MINOR, IDX_MINOR).astype(jnp.int32)
    out = _make_kernel(total, table.shape[0])(x2d, table)
    return out.reshape(b, s, D_MODEL)
